# Initial kernel scaffold; baseline (speedup 1.0000x reference)
import jax
import jax.numpy as jnp
from jax.experimental import pallas as pl


def kernel(x, adj_indices, adj_values, W, b):
    raise NotImplementedError("write your pallas kernel here")


# speedup vs baseline: 5.4927x; 5.4927x over previous
import functools

import jax
import jax.numpy as jnp
from jax import lax
from jax.experimental import pallas as pl
from jax.experimental.pallas import tpu as pltpu
from jax.experimental.pallas import tpu_sc as plsc

N = 10000
E = 320000
D = 128
NC = 2
NS = 16
NW = NC * NS
C = 128
NCHUNKS = E // C
ROWS_PER_TILE = N // NS


def _sc_agg_body(x_hbm, col_hbm, row_hbm, val_hbm, out_hbm,
                 colv, rowv, valv, rowsbuf, acc, sem):
    cid = lax.axis_index("c")
    sid = lax.axis_index("s")
    wid = sid * NC + cid

    zero16 = jnp.zeros((16,), jnp.float32)

    def _zero_rowsbuf(i, carry):
        for j in range(8):
            rowsbuf[i, pl.ds(j * 16, 16)] = zero16
        return carry

    lax.fori_loop(0, C, _zero_rowsbuf, 0)
    for k in range(5):
        pltpu.sync_copy(rowsbuf.at[pl.ds(0, 125)],
                        acc.at[pl.ds(sid * ROWS_PER_TILE + k * 125, 125)])
    plsc.subcore_barrier()

    nk = (NCHUNKS - wid + NW - 1) // NW

    def _chunk_body(k, carry):
        base = (wid + k * NW) * C
        pltpu.sync_copy(col_hbm.at[pl.ds(base, C)], colv)
        pltpu.sync_copy(row_hbm.at[pl.ds(base, C)], rowv)
        pltpu.sync_copy(val_hbm.at[pl.ds(base, C)], valv)
        pltpu.async_copy(x_hbm.at[colv], rowsbuf, sem).wait()

        def _group_body(g, c2):
            v16 = valv[pl.ds(g * 16, 16)]
            for j in range(16):
                vj = v16[j]
                e = g * 16 + j
                for jj in range(8):
                    sl = pl.ds(jj * 16, 16)
                    rowsbuf[e, sl] = rowsbuf[e, sl] * vj
            return c2

        lax.fori_loop(0, C // 16, _group_body, 0)
        pltpu.sync_copy(rowsbuf, acc.at[rowv], add=True)
        return carry

    lax.fori_loop(0, nk, _chunk_body, 0)

    plsc.subcore_barrier()

    @pl.when(sid == 0)
    def _():
        pltpu.sync_copy(acc, out_hbm.at[cid])


_sc_agg = pl.kernel(
    _sc_agg_body,
    out_type=jax.ShapeDtypeStruct((NC, N, D), jnp.float32),
    mesh=plsc.VectorSubcoreMesh(core_axis_name="c", subcore_axis_name="s"),
    scratch_types=[
        pltpu.VMEM((C,), jnp.int32),
        pltpu.VMEM((C,), jnp.int32),
        pltpu.VMEM((C,), jnp.float32),
        pltpu.VMEM((C, D), jnp.float32),
        pltpu.VMEM_SHARED((N, D), jnp.float32),
        pltpu.SemaphoreType.DMA,
    ],
)


def _tc_combine_body(p_ref, w_ref, b_ref, o_ref):
    s = p_ref[0] + p_ref[1]
    o_ref[...] = (
        jnp.dot(s, w_ref[...], preferred_element_type=jnp.float32) + b_ref[...]
    )


_RB = 1000


@jax.jit
def _tc_combine(partials, Wt, b2):
    return pl.pallas_call(
        _tc_combine_body,
        grid=(N // _RB,),
        in_specs=[
            pl.BlockSpec((NC, _RB, D), lambda i: (0, i, 0)),
            pl.BlockSpec((D, D), lambda i: (0, 0)),
            pl.BlockSpec((1, D), lambda i: (0, 0)),
        ],
        out_specs=pl.BlockSpec((_RB, D), lambda i: (i, 0)),
        out_shape=jax.ShapeDtypeStruct((N, D), jnp.float32),
    )(partials, Wt, b2)


def kernel(x, adj_indices, adj_values, W, b):
    row = adj_indices[0].astype(jnp.int32)
    col = adj_indices[1].astype(jnp.int32)
    partials = _sc_agg(x, col, row, adj_values)
    return _tc_combine(partials, W.T, b.reshape(1, D))
